# baseline (speedup 1.0000x reference)
import dataclasses

import jax
import jax.numpy as jnp
from jax import lax
from jax.experimental import pallas as pl
from jax.experimental.pallas import tpu as pltpu
from jax.experimental.pallas import tpu_sc as plsc

B, D, H, E, NX = 8192, 4096, 64, 8, 256
BC = 512
BT = 128
P = B + E * BT
NB = P // BT
NBE = 128
HD = D // 2

_SC_PARAMS = pltpu.CompilerParams()
if "needs_layout_passes" in pltpu.CompilerParams.__dataclass_fields__:
    _SC_PARAMS = dataclasses.replace(_SC_PARAMS, needs_layout_passes=False)

NC, NS = 2, 16
NW = NC * NS
TPW = B // NW
CHG = 128


def _lane_cumsum8(x):
    for sh in (1, 2, 4):
        x = x + jnp.pad(x, ((0, 0), (sh, 0)))[:, :E]
    return x


def _tail_body(h0_ref, Wc1_ref, bc1_ref, Wc2_ref, bc2_ref, Wc3_ref, bc3_ref,
               modes_ref, rank_ref, counts_ref, be_ref, carry_ref):
    h0 = h0_ref[...]
    h1 = jax.nn.relu(jnp.dot(h0, Wc1_ref[...],
                             preferred_element_type=jnp.float32) + bc1_ref[...])
    h2 = jax.nn.relu(jnp.dot(h1, Wc2_ref[...],
                             preferred_element_type=jnp.float32) + bc2_ref[...])
    logits = jnp.dot(h2, Wc3_ref[...],
                     preferred_element_type=jnp.float32) + bc3_ref[...]
    m = jnp.max(logits, axis=-1, keepdims=True)
    u = jnp.exp(logits - m)
    p = u / jnp.sum(u, axis=-1, keepdims=True)
    pmax = jnp.max(p, axis=-1, keepdims=True)
    iota_e = lax.broadcasted_iota(jnp.int32, (BC, E), 1)
    modes = jnp.min(jnp.where(p == pmax, iota_e, E), axis=-1, keepdims=True)
    modes_ref[...] = modes

    @pl.when(pl.program_id(0) == 0)
    def _():
        carry_ref[...] = jnp.zeros((1, E), jnp.float32)

    carry = carry_ref[...]
    onehot = (modes == iota_e).astype(jnp.float32)
    r = lax.broadcasted_iota(jnp.int32, (BC, BC), 0)
    c = lax.broadcasted_iota(jnp.int32, (BC, BC), 1)
    tri = (c <= r).astype(jnp.float32)
    incl = jnp.dot(tri, onehot, preferred_element_type=jnp.float32)
    rank = jnp.sum(onehot * (incl + carry - 1.0), axis=-1, keepdims=True)
    rank_ref[...] = rank.astype(jnp.int32)
    carry_new = carry + incl[BC - 1:BC, :]
    carry_ref[...] = carry_new

    counts_ref[...] = jnp.pad(carry_new.astype(jnp.int32), ((0, 0), (0, E)))
    nb = jnp.floor((carry_new + (BT - 1)) / BT)
    cuminc = _lane_cumsum8(nb).astype(jnp.int32)
    blk = lax.broadcasted_iota(jnp.int32, (NBE, E), 0)
    cb = jnp.broadcast_to(cuminc, (NBE, E))
    be = jnp.sum((blk >= cb).astype(jnp.int32), axis=-1, keepdims=True)
    be_ref[...] = jnp.minimum(be, E - 1)


def _tail_route(h0c, Wc1, bc1, Wc2, bc2, Wc3, bc3):
    return pl.pallas_call(
        _tail_body,
        grid=(B // BC,),
        in_specs=[
            pl.BlockSpec((BC, H), lambda i: (i, 0)),
            pl.BlockSpec((H, H), lambda i: (0, 0)),
            pl.BlockSpec((1, H), lambda i: (0, 0)),
            pl.BlockSpec((H, H), lambda i: (0, 0)),
            pl.BlockSpec((1, H), lambda i: (0, 0)),
            pl.BlockSpec((H, E), lambda i: (0, 0)),
            pl.BlockSpec((1, E), lambda i: (0, 0)),
        ],
        out_specs=[
            pl.BlockSpec((BC, 1), lambda i: (i, 0)),
            pl.BlockSpec((BC, 1), lambda i: (i, 0)),
            pl.BlockSpec((1, 2 * E), lambda i: (0, 0)),
            pl.BlockSpec((NBE, 1), lambda i: (0, 0)),
        ],
        out_shape=[
            jax.ShapeDtypeStruct((B, 1), jnp.int32),
            jax.ShapeDtypeStruct((B, 1), jnp.int32),
            jax.ShapeDtypeStruct((1, 2 * E), jnp.int32),
            jax.ShapeDtypeStruct((NBE, 1), jnp.int32),
        ],
        scratch_shapes=[pltpu.VMEM((1, E), jnp.float32)],
    )(h0c, Wc1, bc1.reshape(1, H), Wc2, bc2.reshape(1, H), Wc3,
      bc3.reshape(1, E))


def _sc_pos_prologue(modes_hbm, rank_hbm, counts_hbm, modes_t, rank_t, pos_t,
                     gstart_v, cnt_v, base):
    pltpu.sync_copy(modes_hbm.at[pl.ds(base, TPW)], modes_t)
    pltpu.sync_copy(rank_hbm.at[pl.ds(base, TPW)], rank_t)
    pltpu.sync_copy(counts_hbm, cnt_v)
    cnt = cnt_v[...]
    nb = (cnt + (BT - 1)) >> 7
    cum = plsc.cumsum(nb)
    gstart_v[...] = (cum - nb) << 7

    @pl.loop(0, TPW, step=16)
    def _(t):
        mv = modes_t[pl.ds(t, 16)]
        gs = plsc.load_gather(gstart_v, [mv])
        pos_t[pl.ds(t, 16)] = rank_t[pl.ds(t, 16)] + gs


CH = 16


def _dispatch(obs, modes, rank, counts):
    mesh = plsc.VectorSubcoreMesh(core_axis_name="c", subcore_axis_name="s")

    @pl.kernel(
        out_type=jax.ShapeDtypeStruct((P, D), jnp.float32),
        mesh=mesh,
        compiler_params=_SC_PARAMS,
        scratch_types=[
            pltpu.VMEM((TPW,), jnp.int32),
            pltpu.VMEM((TPW,), jnp.int32),
            pltpu.VMEM((TPW,), jnp.int32),
            pltpu.VMEM((16,), jnp.int32),
            pltpu.VMEM((16,), jnp.int32),
            pltpu.VMEM((CH, D), jnp.float32),
            pltpu.SemaphoreType.DMA,
        ],
    )
    def k(obs_hbm, modes_hbm, rank_hbm, counts_hbm, xs_hbm,
          modes_t, rank_t, pos_t, gstart_v, cnt_v, buf, sem):
        wid = lax.axis_index("s") * NC + lax.axis_index("c")
        base = wid * TPW
        _sc_pos_prologue(modes_hbm, rank_hbm, counts_hbm, modes_t, rank_t,
                         pos_t, gstart_v, cnt_v, base)

        @pl.loop(0, TPW, step=CH)
        def _(c):
            pv = pos_t[pl.ds(c, CH)]
            pltpu.sync_copy(obs_hbm.at[pl.ds(base + c, CH)], buf)
            pltpu.async_copy(buf, xs_hbm.at[pv], sem).wait()

    return k(obs, modes, rank, counts)


def _bdot(a, b):
    return jnp.dot(a.astype(jnp.bfloat16), b.astype(jnp.bfloat16),
                   preferred_element_type=jnp.float32)


def _expert_body(be_ref, x_ref, W0_ref, b0_ref, W1_ref, b1_ref, W2_ref,
                 b2_ref, W3_ref, b3_ref, y_ref):
    x = x_ref[...]
    h = jax.nn.relu(_bdot(x, W0_ref[0]) + b0_ref[0])
    h = jax.nn.relu(_bdot(h, W1_ref[0]) + b1_ref[0])
    h = jax.nn.relu(_bdot(h, W2_ref[0]) + b2_ref[0])
    y_ref[...] = _bdot(h, W3_ref[0]) + b3_ref[0]


def _experts(block_expert, x_sorted, We0, be0, We1, be1, We2, be2, We3, be3):
    grid_spec = pltpu.PrefetchScalarGridSpec(
        num_scalar_prefetch=1,
        grid=(NB,),
        in_specs=[
            pl.BlockSpec((BT, D), lambda i, be: (i, 0)),
            pl.BlockSpec((1, D, H), lambda i, be: (be[i], 0, 0)),
            pl.BlockSpec((1, 1, H), lambda i, be: (be[i], 0, 0)),
            pl.BlockSpec((1, H, H), lambda i, be: (be[i], 0, 0)),
            pl.BlockSpec((1, 1, H), lambda i, be: (be[i], 0, 0)),
            pl.BlockSpec((1, H, H), lambda i, be: (be[i], 0, 0)),
            pl.BlockSpec((1, 1, H), lambda i, be: (be[i], 0, 0)),
            pl.BlockSpec((1, H, NX), lambda i, be: (be[i], 0, 0)),
            pl.BlockSpec((1, 1, NX), lambda i, be: (be[i], 0, 0)),
        ],
        out_specs=pl.BlockSpec((BT, NX), lambda i, be: (i, 0)),
    )
    return pl.pallas_call(
        _expert_body,
        grid_spec=grid_spec,
        out_shape=jax.ShapeDtypeStruct((P, NX), jnp.float32),
    )(block_expert, x_sorted, We0, be0.reshape(E, 1, H), We1,
      be1.reshape(E, 1, H), We2, be2.reshape(E, 1, H), We3,
      be3.reshape(E, 1, NX))


def _combine(y_sorted, modes, rank, counts):
    mesh = plsc.VectorSubcoreMesh(core_axis_name="c", subcore_axis_name="s")

    @pl.kernel(
        out_type=jax.ShapeDtypeStruct((B, NX), jnp.float32),
        mesh=mesh,
        compiler_params=_SC_PARAMS,
        scratch_types=[
            pltpu.VMEM((TPW,), jnp.int32),
            pltpu.VMEM((TPW,), jnp.int32),
            pltpu.VMEM((TPW,), jnp.int32),
            pltpu.VMEM((16,), jnp.int32),
            pltpu.VMEM((16,), jnp.int32),
            pltpu.VMEM((CHG, NX), jnp.float32),
            pltpu.VMEM((CHG, NX), jnp.float32),
            pltpu.SemaphoreType.DMA,
            pltpu.SemaphoreType.DMA,
        ],
    )
    def k(ys_hbm, modes_hbm, rank_hbm, counts_hbm, out_hbm,
          modes_t, rank_t, pos_t, gstart_v, cnt_v, rows_a, rows_b,
          sem_a, sem_b):
        wid = lax.axis_index("s") * NC + lax.axis_index("c")
        base = wid * TPW
        _sc_pos_prologue(modes_hbm, rank_hbm, counts_hbm, modes_t, rank_t,
                         pos_t, gstart_v, cnt_v, base)

        pltpu.async_copy(ys_hbm.at[pos_t.at[pl.ds(0, CHG)]], rows_a, sem_a)
        pltpu.async_copy(ys_hbm.at[pos_t.at[pl.ds(CHG, CHG)]], rows_b, sem_b)
        pltpu.make_async_copy(ys_hbm.at[pos_t.at[pl.ds(0, CHG)]], rows_a,
                              sem_a).wait()
        pltpu.sync_copy(rows_a, out_hbm.at[pl.ds(base, CHG)])
        pltpu.make_async_copy(ys_hbm.at[pos_t.at[pl.ds(CHG, CHG)]], rows_b,
                              sem_b).wait()
        pltpu.sync_copy(rows_b, out_hbm.at[pl.ds(base + CHG, CHG)])

    return k(y_sorted, modes, rank, counts)


def kernel(obs, Wc0, bc0, Wc1, bc1, Wc2, bc2, Wc3, bc3,
           We0, be0, We1, be1, We2, be2, We3, be3):
    h0c = jax.nn.relu(obs @ Wc0 + bc0)
    modes, rank, counts = None, None, None
    modes2, rank2, counts16, block_expert = _tail_route(
        h0c, Wc1, bc1, Wc2, bc2, Wc3, bc3)
    modes = modes2.reshape(B)
    rank = rank2.reshape(B)
    counts = counts16.reshape(2 * E)
    x_sorted = _dispatch(obs, modes, rank, counts)
    y_sorted = _experts(block_expert.reshape(NBE), x_sorted,
                        We0, be0, We1, be1, We2, be2, We3, be3)
    return _combine(y_sorted, modes, rank, counts)


# speedup vs baseline: 1.0506x; 1.0506x over previous
import dataclasses

import jax
import jax.numpy as jnp
from jax import lax
from jax.experimental import pallas as pl
from jax.experimental.pallas import tpu as pltpu
from jax.experimental.pallas import tpu_sc as plsc

B, D, H, E, NX = 8192, 4096, 64, 8, 256
BC = 512
BT = 128
P = B + E * BT
NB = P // BT
NBE = 128
HD = D // 2

_SC_PARAMS = pltpu.CompilerParams()
if "needs_layout_passes" in pltpu.CompilerParams.__dataclass_fields__:
    _SC_PARAMS = dataclasses.replace(_SC_PARAMS, needs_layout_passes=False)

NC, NS = 2, 16
NW = NC * NS
TPW = B // NW
CHG = 128


def _lane_cumsum8(x):
    for sh in (1, 2, 4):
        x = x + jnp.pad(x, ((0, 0), (sh, 0)))[:, :E]
    return x


def _tail_body(h0_ref, Wc1_ref, bc1_ref, Wc2_ref, bc2_ref, Wc3_ref, bc3_ref,
               modes_ref, rank_ref, counts_ref, be_ref, carry_ref):
    h0 = h0_ref[...]
    h1 = jax.nn.relu(jnp.dot(h0, Wc1_ref[...],
                             preferred_element_type=jnp.float32) + bc1_ref[...])
    h2 = jax.nn.relu(jnp.dot(h1, Wc2_ref[...],
                             preferred_element_type=jnp.float32) + bc2_ref[...])
    logits = jnp.dot(h2, Wc3_ref[...],
                     preferred_element_type=jnp.float32) + bc3_ref[...]
    m = jnp.max(logits, axis=-1, keepdims=True)
    u = jnp.exp(logits - m)
    p = u / jnp.sum(u, axis=-1, keepdims=True)
    pmax = jnp.max(p, axis=-1, keepdims=True)
    iota_e = lax.broadcasted_iota(jnp.int32, (BC, E), 1)
    modes = jnp.min(jnp.where(p == pmax, iota_e, E), axis=-1, keepdims=True)
    modes_ref[...] = modes

    @pl.when(pl.program_id(0) == 0)
    def _():
        carry_ref[...] = jnp.zeros((1, E), jnp.float32)

    carry = carry_ref[...]
    onehot = (modes == iota_e).astype(jnp.float32)
    r = lax.broadcasted_iota(jnp.int32, (BC, BC), 0)
    c = lax.broadcasted_iota(jnp.int32, (BC, BC), 1)
    tri = (c <= r).astype(jnp.float32)
    incl = jnp.dot(tri, onehot, preferred_element_type=jnp.float32)
    rank = jnp.sum(onehot * (incl + carry - 1.0), axis=-1, keepdims=True)
    rank_ref[...] = rank.astype(jnp.int32)
    carry_new = carry + incl[BC - 1:BC, :]
    carry_ref[...] = carry_new

    counts_ref[...] = jnp.pad(carry_new.astype(jnp.int32), ((0, 0), (0, E)))
    nb = jnp.floor((carry_new + (BT - 1)) / BT)
    cuminc = _lane_cumsum8(nb).astype(jnp.int32)
    blk = lax.broadcasted_iota(jnp.int32, (NBE, E), 0)
    cb = jnp.broadcast_to(cuminc, (NBE, E))
    be = jnp.sum((blk >= cb).astype(jnp.int32), axis=-1, keepdims=True)
    be_ref[...] = jnp.minimum(be, E - 1)


def _tail_route(h0c, Wc1, bc1, Wc2, bc2, Wc3, bc3):
    return pl.pallas_call(
        _tail_body,
        grid=(B // BC,),
        in_specs=[
            pl.BlockSpec((BC, H), lambda i: (i, 0)),
            pl.BlockSpec((H, H), lambda i: (0, 0)),
            pl.BlockSpec((1, H), lambda i: (0, 0)),
            pl.BlockSpec((H, H), lambda i: (0, 0)),
            pl.BlockSpec((1, H), lambda i: (0, 0)),
            pl.BlockSpec((H, E), lambda i: (0, 0)),
            pl.BlockSpec((1, E), lambda i: (0, 0)),
        ],
        out_specs=[
            pl.BlockSpec((BC, 1), lambda i: (i, 0)),
            pl.BlockSpec((BC, 1), lambda i: (i, 0)),
            pl.BlockSpec((1, 2 * E), lambda i: (0, 0)),
            pl.BlockSpec((NBE, 1), lambda i: (0, 0)),
        ],
        out_shape=[
            jax.ShapeDtypeStruct((B, 1), jnp.int32),
            jax.ShapeDtypeStruct((B, 1), jnp.int32),
            jax.ShapeDtypeStruct((1, 2 * E), jnp.int32),
            jax.ShapeDtypeStruct((NBE, 1), jnp.int32),
        ],
        scratch_shapes=[pltpu.VMEM((1, E), jnp.float32)],
    )(h0c, Wc1, bc1.reshape(1, H), Wc2, bc2.reshape(1, H), Wc3,
      bc3.reshape(1, E))


def _sc_pos_prologue(modes_hbm, rank_hbm, counts_hbm, modes_t, rank_t, pos_t,
                     gstart_v, cnt_v, base):
    pltpu.sync_copy(modes_hbm.at[pl.ds(base, TPW)], modes_t)
    pltpu.sync_copy(rank_hbm.at[pl.ds(base, TPW)], rank_t)
    pltpu.sync_copy(counts_hbm, cnt_v)
    cnt = cnt_v[...]
    nb = (cnt + (BT - 1)) >> 7
    cum = plsc.cumsum(nb)
    gstart_v[...] = (cum - nb) << 7

    @pl.loop(0, TPW, step=16)
    def _(t):
        mv = modes_t[pl.ds(t, 16)]
        gs = plsc.load_gather(gstart_v, [mv])
        pos_t[pl.ds(t, 16)] = rank_t[pl.ds(t, 16)] + gs


CH = 16


def _dispatch(obs, modes, rank, counts):
    mesh = plsc.VectorSubcoreMesh(core_axis_name="c", subcore_axis_name="s")

    @pl.kernel(
        out_type=jax.ShapeDtypeStruct((P, D), jnp.float32),
        mesh=mesh,
        compiler_params=_SC_PARAMS,
        scratch_types=[
            pltpu.VMEM((TPW,), jnp.int32),
            pltpu.VMEM((TPW,), jnp.int32),
            pltpu.VMEM((TPW,), jnp.int32),
            pltpu.VMEM((16,), jnp.int32),
            pltpu.VMEM((16,), jnp.int32),
            pltpu.VMEM((CH, D), jnp.float32),
            pltpu.SemaphoreType.DMA,
        ],
    )
    def k(obs_hbm, modes_hbm, rank_hbm, counts_hbm, xs_hbm,
          modes_t, rank_t, pos_t, gstart_v, cnt_v, buf, sem):
        wid = lax.axis_index("s") * NC + lax.axis_index("c")
        base = wid * TPW
        _sc_pos_prologue(modes_hbm, rank_hbm, counts_hbm, modes_t, rank_t,
                         pos_t, gstart_v, cnt_v, base)

        @pl.loop(0, TPW, step=CH)
        def _(c):
            pv = pos_t[pl.ds(c, CH)]
            pltpu.sync_copy(obs_hbm.at[pl.ds(base + c, CH)], buf)
            pltpu.async_copy(buf, xs_hbm.at[pv], sem).wait()

    return k(obs, modes, rank, counts)


def _bdot(a, b):
    return jnp.dot(a.astype(jnp.bfloat16), b.astype(jnp.bfloat16),
                   preferred_element_type=jnp.float32)


def _expert_body(be_ref, x_ref, W0_ref, b0_ref, W1_ref, b1_ref, W2_ref,
                 b2_ref, W3_ref, b3_ref, y_ref):
    x = x_ref[...]
    h = jax.nn.relu(_bdot(x, W0_ref[0]) + b0_ref[0])
    h = jax.nn.relu(_bdot(h, W1_ref[0]) + b1_ref[0])
    h = jax.nn.relu(_bdot(h, W2_ref[0]) + b2_ref[0])
    y_ref[...] = _bdot(h, W3_ref[0]) + b3_ref[0]


def _experts(block_expert, x_sorted, We0, be0, We1, be1, We2, be2, We3, be3):
    grid_spec = pltpu.PrefetchScalarGridSpec(
        num_scalar_prefetch=1,
        grid=(NB,),
        in_specs=[
            pl.BlockSpec((BT, D), lambda i, be: (i, 0)),
            pl.BlockSpec((1, D, H), lambda i, be: (be[i], 0, 0)),
            pl.BlockSpec((1, 1, H), lambda i, be: (be[i], 0, 0)),
            pl.BlockSpec((1, H, H), lambda i, be: (be[i], 0, 0)),
            pl.BlockSpec((1, 1, H), lambda i, be: (be[i], 0, 0)),
            pl.BlockSpec((1, H, H), lambda i, be: (be[i], 0, 0)),
            pl.BlockSpec((1, 1, H), lambda i, be: (be[i], 0, 0)),
            pl.BlockSpec((1, H, NX), lambda i, be: (be[i], 0, 0)),
            pl.BlockSpec((1, 1, NX), lambda i, be: (be[i], 0, 0)),
        ],
        out_specs=pl.BlockSpec((BT, NX), lambda i, be: (i, 0)),
    )
    return pl.pallas_call(
        _expert_body,
        grid_spec=grid_spec,
        out_shape=jax.ShapeDtypeStruct((P, NX), jnp.float32),
    )(block_expert, x_sorted, We0, be0.reshape(E, 1, H), We1,
      be1.reshape(E, 1, H), We2, be2.reshape(E, 1, H), We3,
      be3.reshape(E, 1, NX))


def _combine(y_sorted, modes, rank, counts):
    mesh = plsc.VectorSubcoreMesh(core_axis_name="c", subcore_axis_name="s")

    @pl.kernel(
        out_type=jax.ShapeDtypeStruct((B, NX), jnp.float32),
        mesh=mesh,
        compiler_params=_SC_PARAMS,
        scratch_types=[
            pltpu.VMEM((TPW,), jnp.int32),
            pltpu.VMEM((TPW,), jnp.int32),
            pltpu.VMEM((TPW,), jnp.int32),
            pltpu.VMEM((16,), jnp.int32),
            pltpu.VMEM((16,), jnp.int32),
            pltpu.VMEM((CHG, NX), jnp.float32),
            pltpu.VMEM((CHG, NX), jnp.float32),
            pltpu.SemaphoreType.DMA,
            pltpu.SemaphoreType.DMA,
        ],
    )
    def k(ys_hbm, modes_hbm, rank_hbm, counts_hbm, out_hbm,
          modes_t, rank_t, pos_t, gstart_v, cnt_v, rows_a, rows_b,
          sem_a, sem_b):
        wid = lax.axis_index("s") * NC + lax.axis_index("c")
        base = wid * TPW
        _sc_pos_prologue(modes_hbm, rank_hbm, counts_hbm, modes_t, rank_t,
                         pos_t, gstart_v, cnt_v, base)

        pltpu.async_copy(ys_hbm.at[pos_t.at[pl.ds(0, CHG)]], rows_a, sem_a)
        pltpu.async_copy(ys_hbm.at[pos_t.at[pl.ds(CHG, CHG)]], rows_b, sem_b)
        pltpu.make_async_copy(ys_hbm.at[pos_t.at[pl.ds(0, CHG)]], rows_a,
                              sem_a).wait()
        pltpu.sync_copy(rows_a, out_hbm.at[pl.ds(base, CHG)])
        pltpu.make_async_copy(ys_hbm.at[pos_t.at[pl.ds(CHG, CHG)]], rows_b,
                              sem_b).wait()
        pltpu.sync_copy(rows_b, out_hbm.at[pl.ds(base + CHG, CHG)])

    return k(y_sorted, modes, rank, counts)


def kernel(obs, Wc0, bc0, Wc1, bc1, Wc2, bc2, Wc3, bc3,
           We0, be0, We1, be1, We2, be2, We3, be3):
    h0c = jax.nn.relu(obs @ Wc0 + bc0)
    modes, rank, counts = None, None, None
    modes2, rank2, counts16, block_expert = _tail_route(
        h0c, Wc1, bc1, Wc2, bc2, Wc3, bc3)
    modes = modes2.reshape(B)
    rank = rank2.reshape(B)
    counts = counts16.reshape(2 * E)
    x_sorted = _dispatch(obs, modes, rank, counts)
    y_sorted = _experts(block_expert.reshape(NBE), x_sorted,
                        We0, be0, We1, be1, We2, be2, We3, be3)
    return y_sorted
    return _combine(y_sorted, modes, rank, counts)


# speedup vs baseline: 1.0881x; 1.0357x over previous
import dataclasses

import jax
import jax.numpy as jnp
from jax import lax
from jax.experimental import pallas as pl
from jax.experimental.pallas import tpu as pltpu
from jax.experimental.pallas import tpu_sc as plsc

B, D, H, E, NX = 8192, 4096, 64, 8, 256
BC = 512
BT = 512
P = B + E * BT
NB = P // BT
NBE = 128
HD = D // 2

_SC_PARAMS = pltpu.CompilerParams()
if "needs_layout_passes" in pltpu.CompilerParams.__dataclass_fields__:
    _SC_PARAMS = dataclasses.replace(_SC_PARAMS, needs_layout_passes=False)

NC, NS = 2, 16
NW = NC * NS
TPW = B // NW
CHG = 128


def _lane_cumsum8(x):
    for sh in (1, 2, 4):
        x = x + jnp.pad(x, ((0, 0), (sh, 0)))[:, :E]
    return x


def _tail_body(h0_ref, Wc1_ref, bc1_ref, Wc2_ref, bc2_ref, Wc3_ref, bc3_ref,
               modes_ref, rank_ref, counts_ref, be_ref, carry_ref):
    h0 = h0_ref[...]
    h1 = jax.nn.relu(jnp.dot(h0, Wc1_ref[...],
                             preferred_element_type=jnp.float32) + bc1_ref[...])
    h2 = jax.nn.relu(jnp.dot(h1, Wc2_ref[...],
                             preferred_element_type=jnp.float32) + bc2_ref[...])
    logits = jnp.dot(h2, Wc3_ref[...],
                     preferred_element_type=jnp.float32) + bc3_ref[...]
    m = jnp.max(logits, axis=-1, keepdims=True)
    u = jnp.exp(logits - m)
    p = u / jnp.sum(u, axis=-1, keepdims=True)
    pmax = jnp.max(p, axis=-1, keepdims=True)
    iota_e = lax.broadcasted_iota(jnp.int32, (BC, E), 1)
    modes = jnp.min(jnp.where(p == pmax, iota_e, E), axis=-1, keepdims=True)
    modes_ref[...] = modes

    @pl.when(pl.program_id(0) == 0)
    def _():
        carry_ref[...] = jnp.zeros((1, E), jnp.float32)

    carry = carry_ref[...]
    onehot = (modes == iota_e).astype(jnp.float32)
    r = lax.broadcasted_iota(jnp.int32, (BC, BC), 0)
    c = lax.broadcasted_iota(jnp.int32, (BC, BC), 1)
    tri = (c <= r).astype(jnp.float32)
    incl = jnp.dot(tri, onehot, preferred_element_type=jnp.float32)
    rank = jnp.sum(onehot * (incl + carry - 1.0), axis=-1, keepdims=True)
    rank_ref[...] = rank.astype(jnp.int32)
    carry_new = carry + incl[BC - 1:BC, :]
    carry_ref[...] = carry_new

    counts_ref[...] = jnp.pad(carry_new.astype(jnp.int32), ((0, 0), (0, E)))
    nb = jnp.floor((carry_new + (BT - 1)) / BT)
    cuminc = _lane_cumsum8(nb).astype(jnp.int32)
    blk = lax.broadcasted_iota(jnp.int32, (NBE, E), 0)
    cb = jnp.broadcast_to(cuminc, (NBE, E))
    be = jnp.sum((blk >= cb).astype(jnp.int32), axis=-1, keepdims=True)
    be_ref[...] = jnp.minimum(be, E - 1)


def _tail_route(h0c, Wc1, bc1, Wc2, bc2, Wc3, bc3):
    return pl.pallas_call(
        _tail_body,
        grid=(B // BC,),
        in_specs=[
            pl.BlockSpec((BC, H), lambda i: (i, 0)),
            pl.BlockSpec((H, H), lambda i: (0, 0)),
            pl.BlockSpec((1, H), lambda i: (0, 0)),
            pl.BlockSpec((H, H), lambda i: (0, 0)),
            pl.BlockSpec((1, H), lambda i: (0, 0)),
            pl.BlockSpec((H, E), lambda i: (0, 0)),
            pl.BlockSpec((1, E), lambda i: (0, 0)),
        ],
        out_specs=[
            pl.BlockSpec((BC, 1), lambda i: (i, 0)),
            pl.BlockSpec((BC, 1), lambda i: (i, 0)),
            pl.BlockSpec((1, 2 * E), lambda i: (0, 0)),
            pl.BlockSpec((NBE, 1), lambda i: (0, 0)),
        ],
        out_shape=[
            jax.ShapeDtypeStruct((B, 1), jnp.int32),
            jax.ShapeDtypeStruct((B, 1), jnp.int32),
            jax.ShapeDtypeStruct((1, 2 * E), jnp.int32),
            jax.ShapeDtypeStruct((NBE, 1), jnp.int32),
        ],
        scratch_shapes=[pltpu.VMEM((1, E), jnp.float32)],
    )(h0c, Wc1, bc1.reshape(1, H), Wc2, bc2.reshape(1, H), Wc3,
      bc3.reshape(1, E))


def _sc_pos_prologue(modes_hbm, rank_hbm, counts_hbm, modes_t, rank_t, pos_t,
                     gstart_v, cnt_v, base):
    pltpu.sync_copy(modes_hbm.at[pl.ds(base, TPW)], modes_t)
    pltpu.sync_copy(rank_hbm.at[pl.ds(base, TPW)], rank_t)
    pltpu.sync_copy(counts_hbm, cnt_v)
    cnt = cnt_v[...]
    nb = (cnt + (BT - 1)) >> 9
    cum = plsc.cumsum(nb)
    gstart_v[...] = (cum - nb) << 9

    @pl.loop(0, TPW, step=16)
    def _(t):
        mv = modes_t[pl.ds(t, 16)]
        gs = plsc.load_gather(gstart_v, [mv])
        pos_t[pl.ds(t, 16)] = rank_t[pl.ds(t, 16)] + gs


CH = 16


def _dispatch(obs, modes, rank, counts):
    mesh = plsc.VectorSubcoreMesh(core_axis_name="c", subcore_axis_name="s")

    @pl.kernel(
        out_type=jax.ShapeDtypeStruct((P, D), jnp.float32),
        mesh=mesh,
        compiler_params=_SC_PARAMS,
        scratch_types=[
            pltpu.VMEM((TPW,), jnp.int32),
            pltpu.VMEM((TPW,), jnp.int32),
            pltpu.VMEM((TPW,), jnp.int32),
            pltpu.VMEM((16,), jnp.int32),
            pltpu.VMEM((16,), jnp.int32),
            pltpu.VMEM((CH, D), jnp.float32),
            pltpu.SemaphoreType.DMA,
        ],
    )
    def k(obs_hbm, modes_hbm, rank_hbm, counts_hbm, xs_hbm,
          modes_t, rank_t, pos_t, gstart_v, cnt_v, buf, sem):
        wid = lax.axis_index("s") * NC + lax.axis_index("c")
        base = wid * TPW
        _sc_pos_prologue(modes_hbm, rank_hbm, counts_hbm, modes_t, rank_t,
                         pos_t, gstart_v, cnt_v, base)

        @pl.loop(0, TPW, step=CH)
        def _(c):
            pv = pos_t[pl.ds(c, CH)]
            pltpu.sync_copy(obs_hbm.at[pl.ds(base + c, CH)], buf)
            pltpu.async_copy(buf, xs_hbm.at[pv], sem).wait()

    return k(obs, modes, rank, counts)


def _bdot(a, b):
    return jnp.dot(a.astype(jnp.bfloat16), b.astype(jnp.bfloat16),
                   preferred_element_type=jnp.float32)


def _expert_body(be_ref, x_ref, W0_ref, b0_ref, W1_ref, b1_ref, W2_ref,
                 b2_ref, W3_ref, b3_ref, y_ref):
    x = x_ref[...]
    h = jax.nn.relu(_bdot(x, W0_ref[0]) + b0_ref[0])
    h = jax.nn.relu(_bdot(h, W1_ref[0]) + b1_ref[0])
    h = jax.nn.relu(_bdot(h, W2_ref[0]) + b2_ref[0])
    y_ref[...] = _bdot(h, W3_ref[0]) + b3_ref[0]


def _experts(block_expert, x_sorted, We0, be0, We1, be1, We2, be2, We3, be3):
    grid_spec = pltpu.PrefetchScalarGridSpec(
        num_scalar_prefetch=1,
        grid=(NB,),
        in_specs=[
            pl.BlockSpec((BT, D), lambda i, be: (i, 0)),
            pl.BlockSpec((1, D, H), lambda i, be: (be[i], 0, 0)),
            pl.BlockSpec((1, 1, H), lambda i, be: (be[i], 0, 0)),
            pl.BlockSpec((1, H, H), lambda i, be: (be[i], 0, 0)),
            pl.BlockSpec((1, 1, H), lambda i, be: (be[i], 0, 0)),
            pl.BlockSpec((1, H, H), lambda i, be: (be[i], 0, 0)),
            pl.BlockSpec((1, 1, H), lambda i, be: (be[i], 0, 0)),
            pl.BlockSpec((1, H, NX), lambda i, be: (be[i], 0, 0)),
            pl.BlockSpec((1, 1, NX), lambda i, be: (be[i], 0, 0)),
        ],
        out_specs=pl.BlockSpec((BT, NX), lambda i, be: (i, 0)),
    )
    return pl.pallas_call(
        _expert_body,
        grid_spec=grid_spec,
        out_shape=jax.ShapeDtypeStruct((P, NX), jnp.float32),
    )(block_expert, x_sorted, We0, be0.reshape(E, 1, H), We1,
      be1.reshape(E, 1, H), We2, be2.reshape(E, 1, H), We3,
      be3.reshape(E, 1, NX))


def _combine(y_sorted, modes, rank, counts):
    mesh = plsc.VectorSubcoreMesh(core_axis_name="c", subcore_axis_name="s")

    @pl.kernel(
        out_type=jax.ShapeDtypeStruct((B, NX), jnp.float32),
        mesh=mesh,
        compiler_params=_SC_PARAMS,
        scratch_types=[
            pltpu.VMEM((TPW,), jnp.int32),
            pltpu.VMEM((TPW,), jnp.int32),
            pltpu.VMEM((TPW,), jnp.int32),
            pltpu.VMEM((16,), jnp.int32),
            pltpu.VMEM((16,), jnp.int32),
            pltpu.VMEM((CHG, NX), jnp.float32),
            pltpu.VMEM((CHG, NX), jnp.float32),
            pltpu.SemaphoreType.DMA,
            pltpu.SemaphoreType.DMA,
        ],
    )
    def k(ys_hbm, modes_hbm, rank_hbm, counts_hbm, out_hbm,
          modes_t, rank_t, pos_t, gstart_v, cnt_v, rows_a, rows_b,
          sem_a, sem_b):
        wid = lax.axis_index("s") * NC + lax.axis_index("c")
        base = wid * TPW
        _sc_pos_prologue(modes_hbm, rank_hbm, counts_hbm, modes_t, rank_t,
                         pos_t, gstart_v, cnt_v, base)

        pltpu.async_copy(ys_hbm.at[pos_t.at[pl.ds(0, CHG)]], rows_a, sem_a)
        pltpu.async_copy(ys_hbm.at[pos_t.at[pl.ds(CHG, CHG)]], rows_b, sem_b)
        pltpu.make_async_copy(ys_hbm.at[pos_t.at[pl.ds(0, CHG)]], rows_a,
                              sem_a).wait()
        pltpu.sync_copy(rows_a, out_hbm.at[pl.ds(base, CHG)])
        pltpu.make_async_copy(ys_hbm.at[pos_t.at[pl.ds(CHG, CHG)]], rows_b,
                              sem_b).wait()
        pltpu.sync_copy(rows_b, out_hbm.at[pl.ds(base + CHG, CHG)])

    return k(y_sorted, modes, rank, counts)


def kernel(obs, Wc0, bc0, Wc1, bc1, Wc2, bc2, Wc3, bc3,
           We0, be0, We1, be1, We2, be2, We3, be3):
    h0c = jax.nn.relu(obs @ Wc0 + bc0)
    modes, rank, counts = None, None, None
    modes2, rank2, counts16, block_expert = _tail_route(
        h0c, Wc1, bc1, Wc2, bc2, Wc3, bc3)
    modes = modes2.reshape(B)
    rank = rank2.reshape(B)
    counts = counts16.reshape(2 * E)
    x_sorted = _dispatch(obs, modes, rank, counts)
    y_sorted = _experts(block_expert.reshape(NBE), x_sorted,
                        We0, be0, We1, be1, We2, be2, We3, be3)
    return _combine(y_sorted, modes, rank, counts)
